# baseline (speedup 1.0000x reference)
import functools

import jax
import jax.numpy as jnp
from jax import lax
from jax.experimental import pallas as pl
from jax.experimental.pallas import tpu as pltpu
from jax.experimental.pallas import tpu_sc as plsc

N_NODE = 10000
N_EDGE = 320000
N_DIM = 128
E_DIM = 16
M_DIM = 128
G_DIM = 128
T = 3
N_GRAPH = 64
HALF = 256 // 2

NS = 16
EDGES_PER_TILE = N_EDGE // NS
E_CH = 80
E_H = E_CH // 2
N_CHUNKS = EDGES_PER_TILE // E_CH
FLUSH_TILES = 10
FLUSH_ROWS = N_NODE // FLUSH_TILES
ZB = 40

EB = 2560

QSCALE = 2048.0


def _pack_i16(x):
    h2 = x.shape[1] // 2
    q = jnp.clip(jnp.round(x * QSCALE), -32768.0, 32767.0).astype(jnp.int32)
    return (q[:, :h2] & 0xFFFF) | (q[:, h2:] << 16)
NB = 1000


def _c_body(ea_ref, w_ref, b_ref, ca_ref, cb_ref):
    acc = lax.dot_general(ea_ref[...], w_ref[...], (((0,), (0,)), ((), ())),
                          preferred_element_type=jnp.float32)
    acc = acc + b_ref[...]
    ca_ref[...] = _pack_i16(acc[:, :HALF])
    cb_ref[...] = _pack_i16(acc[:, HALF:])


def _build_c(e_attr, w1e, b1):
    return pl.pallas_call(
        _c_body,
        grid=(N_EDGE // EB,),
        in_specs=[
            pl.BlockSpec((E_DIM, EB), lambda i: (0, i)),
            pl.BlockSpec((E_DIM, 2 * HALF), lambda i: (0, 0)),
            pl.BlockSpec((1, 2 * HALF), lambda i: (0, 0)),
        ],
        out_specs=[
            pl.BlockSpec((EB, HALF // 2), lambda i: (i, 0)),
            pl.BlockSpec((EB, HALF // 2), lambda i: (i, 0)),
        ],
        out_shape=[jax.ShapeDtypeStruct((N_EDGE, HALF // 2), jnp.int32)] * 2,
    )(e_attr, w1e, b1)


def _h1_body(h_ref, w_ref, a_ref, b_ref):
    acc = jnp.dot(h_ref[...], w_ref[...], preferred_element_type=jnp.float32)
    acc = acc * QSCALE
    a_ref[...] = acc[:, :HALF]
    b_ref[...] = acc[:, HALF:]


def _build_h1(h_v, w1h):
    return pl.pallas_call(
        _h1_body,
        grid=(N_NODE // NB,),
        in_specs=[
            pl.BlockSpec((NB, N_DIM), lambda i: (i, 0)),
            pl.BlockSpec((N_DIM, 2 * HALF), lambda i: (0, 0)),
        ],
        out_specs=[
            pl.BlockSpec((NB, HALF), lambda i: (i, 0)),
            pl.BlockSpec((NB, HALF), lambda i: (i, 0)),
        ],
        out_shape=[jax.ShapeDtypeStruct((N_NODE, HALF), jnp.float32)] * 2,
    )(h_v, w1h)


def _deg_sc(src):
    mesh = plsc.VectorSubcoreMesh(core_axis_name="c", subcore_axis_name="s")

    @functools.partial(
        pl.kernel,
        out_type=jax.ShapeDtypeStruct((N_NODE, 16), jnp.float32),
        mesh=mesh,
        scratch_types=[
            pltpu.VMEM((E_CH, 16), jnp.float32),
            pltpu.VMEM((ZB, 16), jnp.float32),
            pltpu.VMEM((E_CH,), jnp.int32),
            pltpu.VMEM_SHARED((N_NODE, 16), jnp.float32),
        ],
    )
    def k(src_h, deg_h, ones_v, zbuf, sidx, deg_sh):
        c = lax.axis_index("c")
        s = lax.axis_index("s")

        @pl.when(c == 0)
        def _():
            def fill(i, carry):
                zbuf[i, :] = jnp.zeros((16,), jnp.float32)
                return carry

            lax.fori_loop(0, ZB, fill, 0)

            def fill1(i, carry):
                ones_v[i, :] = jnp.ones((16,), jnp.float32)
                return carry

            lax.fori_loop(0, E_CH, fill1, 0)
            row0 = s * FLUSH_ROWS

            @pl.when(s < FLUSH_TILES)
            def _():
                for part in range(FLUSH_ROWS // ZB):
                    pltpu.sync_copy(zbuf,
                                    deg_sh.at[pl.ds(row0 + part * ZB, ZB), :])

            plsc.subcore_barrier()
            base = s * EDGES_PER_TILE

            def chunk(i, carry):
                pltpu.sync_copy(src_h.at[pl.ds(base + i * E_CH, E_CH)], sidx)
                pltpu.sync_copy(ones_v, deg_sh.at[sidx], add=True)
                return carry

            lax.fori_loop(0, N_CHUNKS, chunk, 0)
            plsc.subcore_barrier()

            @pl.when(s < FLUSH_TILES)
            def _():
                pltpu.sync_copy(deg_sh.at[pl.ds(row0, FLUSH_ROWS), :],
                                deg_h.at[pl.ds(row0, FLUSH_ROWS), :])

    return k(src)


def _message_sc(src, dst, ca, cb, h1a, h1b):
    mesh = plsc.VectorSubcoreMesh(core_axis_name="c", subcore_axis_name="s")

    scratch = [
        pltpu.VMEM((E_H,), jnp.int32),
        pltpu.VMEM((E_H,), jnp.int32),
        pltpu.VMEM((E_H,), jnp.int32),
        pltpu.VMEM((E_H,), jnp.int32),
        pltpu.VMEM((E_CH,), jnp.int32),
        pltpu.VMEM((E_CH,), jnp.int32),
        pltpu.VMEM((E_CH, HALF // 2), jnp.int32),
        pltpu.VMEM((E_CH, HALF // 2), jnp.int32),
        pltpu.VMEM((E_CH, HALF), jnp.float32),
        pltpu.VMEM((E_CH, HALF), jnp.float32),
        pltpu.VMEM((E_H, HALF), jnp.float32),
        pltpu.VMEM_SHARED((N_NODE, HALF), jnp.float32),
    ] + [pltpu.SemaphoreType.DMA] * 8

    @functools.partial(
        pl.kernel,
        out_type=[jax.ShapeDtypeStruct((N_NODE, HALF), jnp.float32)] * 2,
        mesh=mesh,
        scratch_types=scratch,
    )
    def k(src_h, dst_h, ca_h, cb_h, h1a_h, h1b_h, sa_h, sb_h,
          sidxa0, sidxb0, sidxa1, sidxb1, didx0, didx1, cbuf0, cbuf1,
          hbuf0, hbuf1, rbuf, s_sh, sem_si0, sem_si1, sem_di0, sem_di1,
          sem_c0, sem_c1, sem_g0, sem_g1):
        c = lax.axis_index("c")
        s = lax.axis_index("s")
        sidx = ((sidxa0, sidxb0), (sidxa1, sidxb1))
        didx = (didx0, didx1)
        cbufs = (cbuf0, cbuf1)
        hbufs = (hbuf0, hbuf1)
        sem_si = (sem_si0, sem_si1)
        sem_di = (sem_di0, sem_di1)
        sem_c = (sem_c0, sem_c1)
        sem_g = (sem_g0, sem_g1)

        def zrow(i, carry):
            for g in range(HALF // 16):
                rbuf[i, pl.ds(g * 16, 16)] = jnp.zeros((16,), jnp.float32)
            return carry

        lax.fori_loop(0, E_H, zrow, 0)
        row0 = s * FLUSH_ROWS

        @pl.when(s < FLUSH_TILES)
        def _():
            for part in range(FLUSH_ROWS // E_H):
                pltpu.sync_copy(rbuf,
                                s_sh.at[pl.ds(row0 + part * E_H, E_H), :])

        plsc.subcore_barrier()

        base = s * EDGES_PER_TILE

        def edge_loop(c_hbm, h1_hbm):
            def issue_idx(kch, p):
                e0 = base + kch * E_CH
                pltpu.async_copy(src_h.at[pl.ds(e0, E_H)], sidx[p][0],
                                 sem_si[p])
                pltpu.async_copy(src_h.at[pl.ds(e0 + E_H, E_H)], sidx[p][1],
                                 sem_si[p])
                pltpu.async_copy(dst_h.at[pl.ds(e0, E_CH)], didx[p], sem_di[p])

            def wait_idx(p):
                pltpu.make_async_copy(src_h.at[pl.ds(0, E_H)], sidx[p][0],
                                      sem_si[p]).wait()
                pltpu.make_async_copy(src_h.at[pl.ds(0, E_H)], sidx[p][1],
                                      sem_si[p]).wait()
                pltpu.make_async_copy(dst_h.at[pl.ds(0, E_CH)], didx[p],
                                      sem_di[p]).wait()

            def issue_data(kch, p):
                e0 = base + kch * E_CH
                pltpu.async_copy(c_hbm.at[pl.ds(e0, E_CH), :], cbufs[p],
                                 sem_c[p])
                pltpu.async_copy(h1_hbm.at[didx[p]], hbufs[p], sem_g[p])

            def wait_data(p):
                pltpu.make_async_copy(c_hbm.at[pl.ds(0, E_CH), :], cbufs[p],
                                      sem_c[p]).wait()
                pltpu.make_async_copy(h1_hbm.at[didx[p]], hbufs[p],
                                      sem_g[p]).wait()

            issue_idx(0, 0)
            wait_idx(0)
            issue_data(0, 0)
            issue_idx(1, 1)

            def body(i, carry):
                for p in range(2):
                    kch = 2 * i + p
                    q = 1 - p
                    cbuf = cbufs[p]
                    hbuf = hbufs[p]

                    @pl.when(kch + 1 < N_CHUNKS)
                    def _():
                        wait_idx(q)
                        issue_data(kch + 1, q)

                    wait_data(p)

                    for half_ix in range(2):
                        roff = half_ix * E_H

                        @plsc.parallel_loop(0, E_H, unroll=2)
                        def row(rr, roff=roff):
                            r = rr + roff
                            for g in range(HALF // 32):
                                cw = cbuf[r, pl.ds(g * 16, 16)]
                                lo = ((cw << 16) >> 16).astype(jnp.float32)
                                hi = (cw >> 16).astype(jnp.float32)
                                h_lo = hbuf[r, pl.ds(g * 16, 16)]
                                h_hi = hbuf[r, pl.ds(64 + g * 16, 16)]
                                rbuf[rr, pl.ds(g * 32, 16)] = jnp.maximum(
                                    lo + h_lo, 0.0)
                                rbuf[rr, pl.ds(g * 32 + 16, 16)] = jnp.maximum(
                                    hi + h_hi, 0.0)

                        pltpu.sync_copy(rbuf, s_sh.at[sidx[p][half_ix]],
                                        add=True)

                    @pl.when(kch + 2 < N_CHUNKS)
                    def _():
                        issue_idx(kch + 2, p)
                return carry

            lax.fori_loop(0, N_CHUNKS // 2, body, 0)

        @pl.when(c == 0)
        def _():
            edge_loop(ca_h, h1a_h)

        @pl.when(c == 1)
        def _():
            edge_loop(cb_h, h1b_h)

        plsc.subcore_barrier()

        @pl.when(jnp.logical_and(c == 0, s < FLUSH_TILES))
        def _():
            pltpu.sync_copy(s_sh.at[pl.ds(row0, FLUSH_ROWS), :],
                            sa_h.at[pl.ds(row0, FLUSH_ROWS), :])

        @pl.when(jnp.logical_and(c == 1, s < FLUSH_TILES))
        def _():
            pltpu.sync_copy(s_sh.at[pl.ds(row0, FLUSH_ROWS), :],
                            sb_h.at[pl.ds(row0, FLUSH_ROWS), :])

    return k(src, dst, ca, cb, h1a, h1b)


def _update_body(sa_ref, sb_ref, h_ref, deg_ref, w2a_ref, w2b_ref, b2_ref,
                 wih_ref, bih_ref, whh_ref, bhh_ref, out_ref):
    deg = jnp.sum(deg_ref[...], axis=1, keepdims=True)
    m_v = (jnp.dot(sa_ref[...], w2a_ref[...], preferred_element_type=jnp.float32)
           + jnp.dot(sb_ref[...], w2b_ref[...], preferred_element_type=jnp.float32)
           + deg * b2_ref[...])
    h = h_ref[...]
    gi = lax.dot_general(m_v, wih_ref[...], (((1,), (1,)), ((), ())),
                         preferred_element_type=jnp.float32) + bih_ref[...]
    gh = lax.dot_general(h, whh_ref[...], (((1,), (1,)), ((), ())),
                         preferred_element_type=jnp.float32) + bhh_ref[...]
    ir, iz, inn = gi[:, :N_DIM], gi[:, N_DIM:2 * N_DIM], gi[:, 2 * N_DIM:]
    hr, hz, hn = gh[:, :N_DIM], gh[:, N_DIM:2 * N_DIM], gh[:, 2 * N_DIM:]
    r = jax.nn.sigmoid(ir + hr)
    z = jax.nn.sigmoid(iz + hz)
    n = jnp.tanh(inn + r * hn)
    out_ref[...] = (1.0 - z) * n + z * h


def _update(sa, sb, h_v, deg2, w2a, w2b, b2, wih, bih, whh, bhh):
    full = lambda r, c: pl.BlockSpec((r, c), lambda i: (0, 0))
    return pl.pallas_call(
        _update_body,
        grid=(N_NODE // NB,),
        in_specs=[
            pl.BlockSpec((NB, HALF), lambda i: (i, 0)),
            pl.BlockSpec((NB, HALF), lambda i: (i, 0)),
            pl.BlockSpec((NB, N_DIM), lambda i: (i, 0)),
            pl.BlockSpec((NB, 16), lambda i: (i, 0)),
            full(HALF, M_DIM), full(HALF, M_DIM), full(1, M_DIM),
            full(3 * N_DIM, M_DIM), full(1, 3 * N_DIM),
            full(3 * N_DIM, N_DIM), full(1, 3 * N_DIM),
        ],
        out_specs=pl.BlockSpec((NB, N_DIM), lambda i: (i, 0)),
        out_shape=jax.ShapeDtypeStruct((N_NODE, N_DIM), jnp.float32),
    )(sa, sb, h_v, deg2, w2a, w2b, b2, wih, bih, whh, bhh)


def _readout_body(h_ref, h0_ref, g_ref, riw1_ref, rib1_ref, riw2_ref, rib2_ref,
                  rjw1_ref, rjb1_ref, rjw2_ref, rjb2_ref,
                  cw1_ref, cb1_ref, cw2_ref, cb2_ref, out_ref, racc):
    i = pl.program_id(0)

    @pl.when(i == 0)
    def _():
        racc[...] = jnp.zeros_like(racc)

    h = h_ref[...]
    cat = jnp.concatenate([h, h0_ref[...]], axis=1)
    hi = jnp.maximum(jnp.dot(cat, riw1_ref[...],
                             preferred_element_type=jnp.float32) + rib1_ref[...], 0.0)
    i_out = jax.nn.sigmoid(jnp.dot(hi, riw2_ref[...],
                                   preferred_element_type=jnp.float32) + rib2_ref[...])
    hj = jnp.maximum(jnp.dot(h, rjw1_ref[...],
                             preferred_element_type=jnp.float32) + rjb1_ref[...], 0.0)
    j_out = jnp.dot(hj, rjw2_ref[...],
                    preferred_element_type=jnp.float32) + rjb2_ref[...]
    p = i_out * j_out
    cols = lax.broadcasted_iota(jnp.int32, (NB, N_GRAPH), 1)
    onehot = (cols == g_ref[...]).astype(jnp.float32)
    racc[...] += lax.dot_general(onehot, p, (((0,), (0,)), ((), ())),
                                 preferred_element_type=jnp.float32)

    @pl.when(i == N_NODE // NB - 1)
    def _():
        rg = racc[...]
        hc = jnp.maximum(jnp.dot(rg, cw1_ref[...],
                                 preferred_element_type=jnp.float32) + cb1_ref[...], 0.0)
        logit = jnp.dot(hc, cw2_ref[...],
                        preferred_element_type=jnp.float32) + cb2_ref[...]
        out_ref[...] = jax.nn.sigmoid(logit)


def _readout(h_v, h_0, gidx2, riw1, rib1, riw2, rib2, rjw1, rjb1, rjw2, rjb2,
             cw1, cb1, cw2, cb2):
    full = lambda r, c: pl.BlockSpec((r, c), lambda i: (0, 0))
    return pl.pallas_call(
        _readout_body,
        grid=(N_NODE // NB,),
        in_specs=[
            pl.BlockSpec((NB, N_DIM), lambda i: (i, 0)),
            pl.BlockSpec((NB, N_DIM), lambda i: (i, 0)),
            pl.BlockSpec((NB, 1), lambda i: (i, 0)),
            full(2 * N_DIM, 256), full(1, 256), full(256, G_DIM), full(1, G_DIM),
            full(N_DIM, 256), full(1, 256), full(256, G_DIM), full(1, G_DIM),
            full(G_DIM, 128), full(1, 128), full(128, 1), full(1, 1),
        ],
        out_specs=pl.BlockSpec((N_GRAPH, 1), lambda i: (0, 0)),
        out_shape=jax.ShapeDtypeStruct((N_GRAPH, 1), jnp.float32),
        scratch_shapes=[pltpu.VMEM((N_GRAPH, G_DIM), jnp.float32)],
    )(h_v, h_0, gidx2, riw1, rib1, riw2, rib2, rjw1, rjb1, rjw2, rjb2,
      cw1, cb1, cw2, cb2)


def kernel(h_0, E_attr, m_W1, m_b1, m_W2, m_b2, gru_Wih, gru_Whh, gru_bih,
           gru_bhh, ri_W1, ri_b1, ri_W2, ri_b2, rj_W1, rj_b1, rj_W2, rj_b2,
           c_W1, c_b1, c_W2, c_b2, graph_index, E):
    src = E[0]
    dst = E[1]
    w1h = m_W1[:N_DIM]
    w1e = m_W1[N_DIM:]

    ca, cb = _build_c(E_attr, w1e, m_b1.reshape(1, -1))
    deg2 = _deg_sc(src)

    pi = []
    for g in range(HALF // 32):
        pi.extend(16 * g + i for i in range(16))
        pi.extend(64 + 16 * g + i for i in range(16))
    pi = jnp.array(pi, dtype=jnp.int32)
    w2a = m_W2[:HALF][pi] * (1.0 / QSCALE)
    w2b = m_W2[HALF:][pi] * (1.0 / QSCALE)
    b2 = m_b2.reshape(1, -1)
    bih = gru_bih.reshape(1, -1)
    bhh = gru_bhh.reshape(1, -1)

    h_v = h_0
    for _ in range(T):
        h1a, h1b = _build_h1(h_v, w1h)
        sa, sb = _message_sc(src, dst, ca, cb, h1a, h1b)
        h_v = _update(sa, sb, h_v, deg2, w2a, w2b, b2, gru_Wih, bih,
                      gru_Whh, bhh)

    out = _readout(h_v, h_0, graph_index.reshape(N_NODE, 1),
                   ri_W1, ri_b1.reshape(1, -1), ri_W2, ri_b2.reshape(1, -1),
                   rj_W1, rj_b1.reshape(1, -1), rj_W2, rj_b2.reshape(1, -1),
                   c_W1, c_b1.reshape(1, -1), c_W2, c_b2.reshape(1, -1))
    return out.reshape(N_GRAPH)


# speedup vs baseline: 1.1134x; 1.1134x over previous
import functools

import jax
import jax.numpy as jnp
from jax import lax
from jax.experimental import pallas as pl
from jax.experimental.pallas import tpu as pltpu
from jax.experimental.pallas import tpu_sc as plsc

N_NODE = 10000
N_EDGE = 320000
N_DIM = 128
E_DIM = 16
M_DIM = 128
G_DIM = 128
T = 3
N_GRAPH = 64
HALF = 256 // 2

NS = 16
EDGES_PER_TILE = N_EDGE // NS
E_CH = 80
E_H = E_CH // 2
N_CHUNKS = EDGES_PER_TILE // E_CH
FLUSH_TILES = 10
FLUSH_ROWS = N_NODE // FLUSH_TILES
ZB = 40

EB = 2560

QSCALE = 2048.0


def _pack_i16(x):
    h2 = x.shape[1] // 2
    q = jnp.clip(jnp.round(x * QSCALE), -32768.0, 32767.0).astype(jnp.int32)
    return (q[:, :h2] & 0xFFFF) | (q[:, h2:] << 16)
NB = 1000


def _c_body(ea_ref, w_ref, b_ref, ca_ref, cb_ref):
    acc = lax.dot_general(ea_ref[...], w_ref[...], (((0,), (0,)), ((), ())),
                          preferred_element_type=jnp.float32)
    acc = acc + b_ref[...]
    ca_ref[...] = _pack_i16(acc[:, :HALF])
    cb_ref[...] = _pack_i16(acc[:, HALF:])


def _build_c(e_attr, w1e, b1):
    return pl.pallas_call(
        _c_body,
        grid=(N_EDGE // EB,),
        in_specs=[
            pl.BlockSpec((E_DIM, EB), lambda i: (0, i)),
            pl.BlockSpec((E_DIM, 2 * HALF), lambda i: (0, 0)),
            pl.BlockSpec((1, 2 * HALF), lambda i: (0, 0)),
        ],
        out_specs=[
            pl.BlockSpec((EB, HALF // 2), lambda i: (i, 0)),
            pl.BlockSpec((EB, HALF // 2), lambda i: (i, 0)),
        ],
        out_shape=[jax.ShapeDtypeStruct((N_EDGE, HALF // 2), jnp.int32)] * 2,
    )(e_attr, w1e, b1)


def _h1_body(h_ref, w_ref, a_ref, b_ref):
    acc = jnp.dot(h_ref[...], w_ref[...], preferred_element_type=jnp.float32)
    acc = acc * QSCALE
    a_ref[...] = acc[:, :HALF]
    b_ref[...] = acc[:, HALF:]


def _build_h1(h_v, w1h):
    return pl.pallas_call(
        _h1_body,
        grid=(N_NODE // NB,),
        in_specs=[
            pl.BlockSpec((NB, N_DIM), lambda i: (i, 0)),
            pl.BlockSpec((N_DIM, 2 * HALF), lambda i: (0, 0)),
        ],
        out_specs=[
            pl.BlockSpec((NB, HALF), lambda i: (i, 0)),
            pl.BlockSpec((NB, HALF), lambda i: (i, 0)),
        ],
        out_shape=[jax.ShapeDtypeStruct((N_NODE, HALF), jnp.float32)] * 2,
    )(h_v, w1h)


def _deg_sc(src):
    mesh = plsc.VectorSubcoreMesh(core_axis_name="c", subcore_axis_name="s")

    @functools.partial(
        pl.kernel,
        out_type=jax.ShapeDtypeStruct((N_NODE, 16), jnp.float32),
        mesh=mesh,
        scratch_types=[
            pltpu.VMEM((E_CH, 16), jnp.float32),
            pltpu.VMEM((ZB, 16), jnp.float32),
            pltpu.VMEM((E_CH,), jnp.int32),
            pltpu.VMEM_SHARED((N_NODE, 16), jnp.float32),
        ],
    )
    def k(src_h, deg_h, ones_v, zbuf, sidx, deg_sh):
        c = lax.axis_index("c")
        s = lax.axis_index("s")

        @pl.when(c == 0)
        def _():
            def fill(i, carry):
                zbuf[i, :] = jnp.zeros((16,), jnp.float32)
                return carry

            lax.fori_loop(0, ZB, fill, 0)

            def fill1(i, carry):
                ones_v[i, :] = jnp.ones((16,), jnp.float32)
                return carry

            lax.fori_loop(0, E_CH, fill1, 0)
            row0 = s * FLUSH_ROWS

            @pl.when(s < FLUSH_TILES)
            def _():
                for part in range(FLUSH_ROWS // ZB):
                    pltpu.sync_copy(zbuf,
                                    deg_sh.at[pl.ds(row0 + part * ZB, ZB), :])

            plsc.subcore_barrier()
            base = s * EDGES_PER_TILE

            def chunk(i, carry):
                pltpu.sync_copy(src_h.at[pl.ds(base + i * E_CH, E_CH)], sidx)
                pltpu.sync_copy(ones_v, deg_sh.at[sidx], add=True)
                return carry

            lax.fori_loop(0, N_CHUNKS, chunk, 0)
            plsc.subcore_barrier()

            @pl.when(s < FLUSH_TILES)
            def _():
                pltpu.sync_copy(deg_sh.at[pl.ds(row0, FLUSH_ROWS), :],
                                deg_h.at[pl.ds(row0, FLUSH_ROWS), :])

    return k(src)


def _message_sc(src, dst, ca, cb, h1a, h1b):
    mesh = plsc.VectorSubcoreMesh(core_axis_name="c", subcore_axis_name="s")

    scratch = [
        pltpu.VMEM((E_CH,), jnp.int32),
        pltpu.VMEM((E_CH,), jnp.int32),
        pltpu.VMEM((E_CH,), jnp.int32),
        pltpu.VMEM((E_CH,), jnp.int32),
        pltpu.VMEM((E_CH, HALF // 2), jnp.int32),
        pltpu.VMEM((E_CH, HALF // 2), jnp.int32),
        pltpu.VMEM((E_CH, HALF), jnp.float32),
        pltpu.VMEM((E_CH, HALF), jnp.float32),
        pltpu.VMEM_SHARED((N_NODE, HALF), jnp.float32),
    ] + [pltpu.SemaphoreType.DMA] * 10

    @functools.partial(
        pl.kernel,
        out_type=[jax.ShapeDtypeStruct((N_NODE, HALF), jnp.float32)] * 2,
        mesh=mesh,
        scratch_types=scratch,
    )
    def k(src_h, dst_h, ca_h, cb_h, h1a_h, h1b_h, sa_h, sb_h,
          sidx0, sidx1, didx0, didx1, cbuf0, cbuf1, hbuf0, hbuf1, s_sh,
          sem_si0, sem_si1, sem_di0, sem_di1, sem_c0, sem_c1, sem_g0,
          sem_g1, sem_s0, sem_s1):
        c = lax.axis_index("c")
        s = lax.axis_index("s")
        sidx = (sidx0, sidx1)
        didx = (didx0, didx1)
        sem_s = (sem_s0, sem_s1)
        cbufs = (cbuf0, cbuf1)
        hbufs = (hbuf0, hbuf1)
        sem_si = (sem_si0, sem_si1)
        sem_di = (sem_di0, sem_di1)
        sem_c = (sem_c0, sem_c1)
        sem_g = (sem_g0, sem_g1)

        def zrow(i, carry):
            for g in range(HALF // 16):
                hbuf0[i, pl.ds(g * 16, 16)] = jnp.zeros((16,), jnp.float32)
            return carry

        lax.fori_loop(0, E_CH, zrow, 0)
        row0 = s * FLUSH_ROWS

        @pl.when(s < FLUSH_TILES)
        def _():
            for part in range(FLUSH_ROWS // E_CH):
                pltpu.sync_copy(hbuf0,
                                s_sh.at[pl.ds(row0 + part * E_CH, E_CH), :])
            pltpu.sync_copy(
                hbuf0.at[pl.ds(0, E_H), :],
                s_sh.at[pl.ds(row0 + (FLUSH_ROWS // E_CH) * E_CH, E_H), :])

        plsc.subcore_barrier()

        base = s * EDGES_PER_TILE

        def edge_loop(c_hbm, h1_hbm):
            def issue_idx(kch, p):
                e0 = base + kch * E_CH
                pltpu.async_copy(src_h.at[pl.ds(e0, E_CH)], sidx[p], sem_si[p])
                pltpu.async_copy(dst_h.at[pl.ds(e0, E_CH)], didx[p], sem_di[p])

            def wait_idx(p):
                pltpu.make_async_copy(src_h.at[pl.ds(0, E_CH)], sidx[p],
                                      sem_si[p]).wait()
                pltpu.make_async_copy(dst_h.at[pl.ds(0, E_CH)], didx[p],
                                      sem_di[p]).wait()

            def issue_data(kch, p):
                e0 = base + kch * E_CH
                pltpu.async_copy(c_hbm.at[pl.ds(e0, E_CH), :], cbufs[p],
                                 sem_c[p])
                pltpu.async_copy(h1_hbm.at[didx[p]], hbufs[p], sem_g[p])

            def wait_data(p):
                pltpu.make_async_copy(c_hbm.at[pl.ds(0, E_CH), :], cbufs[p],
                                      sem_c[p]).wait()
                pltpu.make_async_copy(h1_hbm.at[didx[p]], hbufs[p],
                                      sem_g[p]).wait()

            issue_idx(0, 0)
            wait_idx(0)
            issue_data(0, 0)
            issue_idx(1, 1)

            def body(i, carry):
                for p in range(2):
                    kch = 2 * i + p
                    q = 1 - p
                    cbuf = cbufs[p]
                    hbuf = hbufs[p]

                    @pl.when(kch + 1 < N_CHUNKS)
                    def _():
                        wait_idx(q)

                        @pl.when(kch >= 1)
                        def _():
                            pltpu.make_async_copy(hbufs[q],
                                                  s_sh.at[sidx[q]],
                                                  sem_s[q]).wait()

                        issue_data(kch + 1, q)

                    wait_data(p)

                    def row(r, carry2):
                        cws = [cbuf[r, pl.ds(g * 16, 16)]
                               for g in range(HALF // 32)]
                        hlos = [hbuf[r, pl.ds(g * 16, 16)]
                                for g in range(HALF // 32)]
                        hhis = [hbuf[r, pl.ds(64 + g * 16, 16)]
                                for g in range(HALF // 32)]
                        for g in range(HALF // 32):
                            lo = ((cws[g] << 16) >> 16).astype(jnp.float32)
                            hi = (cws[g] >> 16).astype(jnp.float32)
                            hbuf[r, pl.ds(g * 32, 16)] = jnp.maximum(
                                lo + hlos[g], 0.0)
                            hbuf[r, pl.ds(g * 32 + 16, 16)] = jnp.maximum(
                                hi + hhis[g], 0.0)
                        return carry2

                    lax.fori_loop(0, E_CH, row, 0)
                    pltpu.async_copy(hbuf, s_sh.at[sidx[p]], sem_s[p],
                                     add=True)

                    @pl.when(kch + 2 < N_CHUNKS)
                    def _():
                        issue_idx(kch + 2, p)
                return carry

            lax.fori_loop(0, N_CHUNKS // 2, body, 0)
            for p in range(2):
                pltpu.make_async_copy(hbufs[p], s_sh.at[sidx[p]],
                                      sem_s[p]).wait()

        @pl.when(c == 0)
        def _():
            edge_loop(ca_h, h1a_h)

        @pl.when(c == 1)
        def _():
            edge_loop(cb_h, h1b_h)

        plsc.subcore_barrier()

        @pl.when(jnp.logical_and(c == 0, s < FLUSH_TILES))
        def _():
            pltpu.sync_copy(s_sh.at[pl.ds(row0, FLUSH_ROWS), :],
                            sa_h.at[pl.ds(row0, FLUSH_ROWS), :])

        @pl.when(jnp.logical_and(c == 1, s < FLUSH_TILES))
        def _():
            pltpu.sync_copy(s_sh.at[pl.ds(row0, FLUSH_ROWS), :],
                            sb_h.at[pl.ds(row0, FLUSH_ROWS), :])

    return k(src, dst, ca, cb, h1a, h1b)


def _update_body(sa_ref, sb_ref, h_ref, deg_ref, w2a_ref, w2b_ref, b2_ref,
                 wih_ref, bih_ref, whh_ref, bhh_ref, out_ref):
    deg = jnp.sum(deg_ref[...], axis=1, keepdims=True)
    m_v = (jnp.dot(sa_ref[...], w2a_ref[...], preferred_element_type=jnp.float32)
           + jnp.dot(sb_ref[...], w2b_ref[...], preferred_element_type=jnp.float32)
           + deg * b2_ref[...])
    h = h_ref[...]
    gi = lax.dot_general(m_v, wih_ref[...], (((1,), (1,)), ((), ())),
                         preferred_element_type=jnp.float32) + bih_ref[...]
    gh = lax.dot_general(h, whh_ref[...], (((1,), (1,)), ((), ())),
                         preferred_element_type=jnp.float32) + bhh_ref[...]
    ir, iz, inn = gi[:, :N_DIM], gi[:, N_DIM:2 * N_DIM], gi[:, 2 * N_DIM:]
    hr, hz, hn = gh[:, :N_DIM], gh[:, N_DIM:2 * N_DIM], gh[:, 2 * N_DIM:]
    r = jax.nn.sigmoid(ir + hr)
    z = jax.nn.sigmoid(iz + hz)
    n = jnp.tanh(inn + r * hn)
    out_ref[...] = (1.0 - z) * n + z * h


def _update(sa, sb, h_v, deg2, w2a, w2b, b2, wih, bih, whh, bhh):
    full = lambda r, c: pl.BlockSpec((r, c), lambda i: (0, 0))
    return pl.pallas_call(
        _update_body,
        grid=(N_NODE // NB,),
        in_specs=[
            pl.BlockSpec((NB, HALF), lambda i: (i, 0)),
            pl.BlockSpec((NB, HALF), lambda i: (i, 0)),
            pl.BlockSpec((NB, N_DIM), lambda i: (i, 0)),
            pl.BlockSpec((NB, 16), lambda i: (i, 0)),
            full(HALF, M_DIM), full(HALF, M_DIM), full(1, M_DIM),
            full(3 * N_DIM, M_DIM), full(1, 3 * N_DIM),
            full(3 * N_DIM, N_DIM), full(1, 3 * N_DIM),
        ],
        out_specs=pl.BlockSpec((NB, N_DIM), lambda i: (i, 0)),
        out_shape=jax.ShapeDtypeStruct((N_NODE, N_DIM), jnp.float32),
    )(sa, sb, h_v, deg2, w2a, w2b, b2, wih, bih, whh, bhh)


def _readout_body(h_ref, h0_ref, g_ref, riw1_ref, rib1_ref, riw2_ref, rib2_ref,
                  rjw1_ref, rjb1_ref, rjw2_ref, rjb2_ref,
                  cw1_ref, cb1_ref, cw2_ref, cb2_ref, out_ref, racc):
    i = pl.program_id(0)

    @pl.when(i == 0)
    def _():
        racc[...] = jnp.zeros_like(racc)

    h = h_ref[...]
    cat = jnp.concatenate([h, h0_ref[...]], axis=1)
    hi = jnp.maximum(jnp.dot(cat, riw1_ref[...],
                             preferred_element_type=jnp.float32) + rib1_ref[...], 0.0)
    i_out = jax.nn.sigmoid(jnp.dot(hi, riw2_ref[...],
                                   preferred_element_type=jnp.float32) + rib2_ref[...])
    hj = jnp.maximum(jnp.dot(h, rjw1_ref[...],
                             preferred_element_type=jnp.float32) + rjb1_ref[...], 0.0)
    j_out = jnp.dot(hj, rjw2_ref[...],
                    preferred_element_type=jnp.float32) + rjb2_ref[...]
    p = i_out * j_out
    cols = lax.broadcasted_iota(jnp.int32, (NB, N_GRAPH), 1)
    onehot = (cols == g_ref[...]).astype(jnp.float32)
    racc[...] += lax.dot_general(onehot, p, (((0,), (0,)), ((), ())),
                                 preferred_element_type=jnp.float32)

    @pl.when(i == N_NODE // NB - 1)
    def _():
        rg = racc[...]
        hc = jnp.maximum(jnp.dot(rg, cw1_ref[...],
                                 preferred_element_type=jnp.float32) + cb1_ref[...], 0.0)
        logit = jnp.dot(hc, cw2_ref[...],
                        preferred_element_type=jnp.float32) + cb2_ref[...]
        out_ref[...] = jax.nn.sigmoid(logit)


def _readout(h_v, h_0, gidx2, riw1, rib1, riw2, rib2, rjw1, rjb1, rjw2, rjb2,
             cw1, cb1, cw2, cb2):
    full = lambda r, c: pl.BlockSpec((r, c), lambda i: (0, 0))
    return pl.pallas_call(
        _readout_body,
        grid=(N_NODE // NB,),
        in_specs=[
            pl.BlockSpec((NB, N_DIM), lambda i: (i, 0)),
            pl.BlockSpec((NB, N_DIM), lambda i: (i, 0)),
            pl.BlockSpec((NB, 1), lambda i: (i, 0)),
            full(2 * N_DIM, 256), full(1, 256), full(256, G_DIM), full(1, G_DIM),
            full(N_DIM, 256), full(1, 256), full(256, G_DIM), full(1, G_DIM),
            full(G_DIM, 128), full(1, 128), full(128, 1), full(1, 1),
        ],
        out_specs=pl.BlockSpec((N_GRAPH, 1), lambda i: (0, 0)),
        out_shape=jax.ShapeDtypeStruct((N_GRAPH, 1), jnp.float32),
        scratch_shapes=[pltpu.VMEM((N_GRAPH, G_DIM), jnp.float32)],
    )(h_v, h_0, gidx2, riw1, rib1, riw2, rib2, rjw1, rjb1, rjw2, rjb2,
      cw1, cb1, cw2, cb2)


def kernel(h_0, E_attr, m_W1, m_b1, m_W2, m_b2, gru_Wih, gru_Whh, gru_bih,
           gru_bhh, ri_W1, ri_b1, ri_W2, ri_b2, rj_W1, rj_b1, rj_W2, rj_b2,
           c_W1, c_b1, c_W2, c_b2, graph_index, E):
    src = E[0]
    dst = E[1]
    w1h = m_W1[:N_DIM]
    w1e = m_W1[N_DIM:]

    ca, cb = _build_c(E_attr, w1e, m_b1.reshape(1, -1))
    deg2 = _deg_sc(src)

    pi = []
    for g in range(HALF // 32):
        pi.extend(16 * g + i for i in range(16))
        pi.extend(64 + 16 * g + i for i in range(16))
    pi = jnp.array(pi, dtype=jnp.int32)
    w2a = m_W2[:HALF][pi] * (1.0 / QSCALE)
    w2b = m_W2[HALF:][pi] * (1.0 / QSCALE)
    b2 = m_b2.reshape(1, -1)
    bih = gru_bih.reshape(1, -1)
    bhh = gru_bhh.reshape(1, -1)

    h_v = h_0
    for _ in range(T):
        h1a, h1b = _build_h1(h_v, w1h)
        sa, sb = _message_sc(src, dst, ca, cb, h1a, h1b)
        h_v = _update(sa, sb, h_v, deg2, w2a, w2b, b2, gru_Wih, bih,
                      gru_Whh, bhh)

    out = _readout(h_v, h_0, graph_index.reshape(N_NODE, 1),
                   ri_W1, ri_b1.reshape(1, -1), ri_W2, ri_b2.reshape(1, -1),
                   rj_W1, rj_b1.reshape(1, -1), rj_W2, rj_b2.reshape(1, -1),
                   c_W1, c_b1.reshape(1, -1), c_W2, c_b2.reshape(1, -1))
    return out.reshape(N_GRAPH)


# speedup vs baseline: 1.1630x; 1.0446x over previous
import functools

import jax
import jax.numpy as jnp
from jax import lax
from jax.experimental import pallas as pl
from jax.experimental.pallas import tpu as pltpu
from jax.experimental.pallas import tpu_sc as plsc

N_NODE = 10000
N_EDGE = 320000
N_DIM = 128
E_DIM = 16
M_DIM = 128
G_DIM = 128
T = 3
N_GRAPH = 64
HALF = 256 // 2

NS = 16
EDGES_PER_TILE = N_EDGE // NS
E_CH = 80
E_H = E_CH // 2
N_CHUNKS = EDGES_PER_TILE // E_CH
FLUSH_TILES = 10
FLUSH_ROWS = N_NODE // FLUSH_TILES
ZB = 40

EB = 2560

QSCALE = 2048.0


def _pack_i16(x):
    h2 = x.shape[1] // 2
    q = jnp.clip(jnp.round(x * QSCALE), -32768.0, 32767.0).astype(jnp.int32)
    return (q[:, :h2] & 0xFFFF) | (q[:, h2:] << 16)
NB = 1000


def _c_body(ea_ref, w_ref, b_ref, ca_ref, cb_ref):
    acc = lax.dot_general(ea_ref[...], w_ref[...], (((0,), (0,)), ((), ())),
                          preferred_element_type=jnp.float32)
    acc = acc + b_ref[...]
    ca_ref[...] = _pack_i16(acc[:, :HALF])
    cb_ref[...] = _pack_i16(acc[:, HALF:])


def _build_c(e_attr, w1e, b1):
    return pl.pallas_call(
        _c_body,
        grid=(N_EDGE // EB,),
        in_specs=[
            pl.BlockSpec((E_DIM, EB), lambda i: (0, i)),
            pl.BlockSpec((E_DIM, 2 * HALF), lambda i: (0, 0)),
            pl.BlockSpec((1, 2 * HALF), lambda i: (0, 0)),
        ],
        out_specs=[
            pl.BlockSpec((EB, HALF // 2), lambda i: (i, 0)),
            pl.BlockSpec((EB, HALF // 2), lambda i: (i, 0)),
        ],
        out_shape=[jax.ShapeDtypeStruct((N_EDGE, HALF // 2), jnp.int32)] * 2,
    )(e_attr, w1e, b1)


def _h1_body(h_ref, w_ref, a_ref, b_ref):
    acc = jnp.dot(h_ref[...], w_ref[...], preferred_element_type=jnp.float32)
    acc = acc * QSCALE
    a_ref[...] = acc[:, :HALF]
    b_ref[...] = acc[:, HALF:]


def _build_h1(h_v, w1h):
    return pl.pallas_call(
        _h1_body,
        grid=(N_NODE // NB,),
        in_specs=[
            pl.BlockSpec((NB, N_DIM), lambda i: (i, 0)),
            pl.BlockSpec((N_DIM, 2 * HALF), lambda i: (0, 0)),
        ],
        out_specs=[
            pl.BlockSpec((NB, HALF), lambda i: (i, 0)),
            pl.BlockSpec((NB, HALF), lambda i: (i, 0)),
        ],
        out_shape=[jax.ShapeDtypeStruct((N_NODE, HALF), jnp.float32)] * 2,
    )(h_v, w1h)


def _deg_sc(src):
    mesh = plsc.VectorSubcoreMesh(core_axis_name="c", subcore_axis_name="s")

    @functools.partial(
        pl.kernel,
        out_type=jax.ShapeDtypeStruct((N_NODE, 16), jnp.float32),
        mesh=mesh,
        scratch_types=[
            pltpu.VMEM((E_CH, 16), jnp.float32),
            pltpu.VMEM((ZB, 16), jnp.float32),
            pltpu.VMEM((E_CH,), jnp.int32),
            pltpu.VMEM((E_CH,), jnp.int32),
            pltpu.VMEM_SHARED((N_NODE, 16), jnp.float32),
            pltpu.SemaphoreType.DMA,
            pltpu.SemaphoreType.DMA,
        ],
    )
    def k(src_h, deg_h, ones_v, zbuf, sidx0, sidx1, deg_sh, sem0, sem1):
        c = lax.axis_index("c")
        s = lax.axis_index("s")
        sidx = (sidx0, sidx1)
        sems = (sem0, sem1)

        @pl.when(c == 0)
        def _():
            def fill(i, carry):
                zbuf[i, :] = jnp.zeros((16,), jnp.float32)
                return carry

            lax.fori_loop(0, ZB, fill, 0)

            def fill1(i, carry):
                ones_v[i, :] = jnp.ones((16,), jnp.float32)
                return carry

            lax.fori_loop(0, E_CH, fill1, 0)
            row0 = s * FLUSH_ROWS

            @pl.when(s < FLUSH_TILES)
            def _():
                for part in range(FLUSH_ROWS // ZB):
                    pltpu.sync_copy(zbuf,
                                    deg_sh.at[pl.ds(row0 + part * ZB, ZB), :])

            plsc.subcore_barrier()
            base = s * EDGES_PER_TILE

            def issue(kch, p):
                pltpu.async_copy(src_h.at[pl.ds(base + kch * E_CH, E_CH)],
                                 sidx[p], sems[p])

            issue(0, 0)
            issue(1, 1)

            def body(i, carry):
                for p in range(2):
                    kch = 2 * i + p
                    pltpu.make_async_copy(src_h.at[pl.ds(0, E_CH)], sidx[p],
                                          sems[p]).wait()
                    pltpu.sync_copy(ones_v, deg_sh.at[sidx[p]], add=True)

                    @pl.when(kch + 2 < N_CHUNKS)
                    def _():
                        issue(kch + 2, p)
                return carry

            lax.fori_loop(0, N_CHUNKS // 2, body, 0)
            plsc.subcore_barrier()

            @pl.when(s < FLUSH_TILES)
            def _():
                pltpu.sync_copy(deg_sh.at[pl.ds(row0, FLUSH_ROWS), :],
                                deg_h.at[pl.ds(row0, FLUSH_ROWS), :])

    return k(src)


def _message_sc(src, dst, ca, cb, h1a, h1b):
    mesh = plsc.VectorSubcoreMesh(core_axis_name="c", subcore_axis_name="s")

    scratch = [
        pltpu.VMEM((E_CH,), jnp.int32),
        pltpu.VMEM((E_CH,), jnp.int32),
        pltpu.VMEM((E_CH,), jnp.int32),
        pltpu.VMEM((E_CH,), jnp.int32),
        pltpu.VMEM((E_CH, HALF // 2), jnp.int32),
        pltpu.VMEM((E_CH, HALF // 2), jnp.int32),
        pltpu.VMEM((E_CH, HALF), jnp.float32),
        pltpu.VMEM((E_CH, HALF), jnp.float32),
        pltpu.VMEM_SHARED((N_NODE, HALF), jnp.float32),
    ] + [pltpu.SemaphoreType.DMA] * 10

    @functools.partial(
        pl.kernel,
        out_type=[jax.ShapeDtypeStruct((N_NODE, HALF), jnp.float32)] * 2,
        mesh=mesh,
        scratch_types=scratch,
    )
    def k(src_h, dst_h, ca_h, cb_h, h1a_h, h1b_h, sa_h, sb_h,
          sidx0, sidx1, didx0, didx1, cbuf0, cbuf1, hbuf0, hbuf1, s_sh,
          sem_si0, sem_si1, sem_di0, sem_di1, sem_c0, sem_c1, sem_g0,
          sem_g1, sem_s0, sem_s1):
        c = lax.axis_index("c")
        s = lax.axis_index("s")
        sidx = (sidx0, sidx1)
        didx = (didx0, didx1)
        sem_s = (sem_s0, sem_s1)
        cbufs = (cbuf0, cbuf1)
        hbufs = (hbuf0, hbuf1)
        sem_si = (sem_si0, sem_si1)
        sem_di = (sem_di0, sem_di1)
        sem_c = (sem_c0, sem_c1)
        sem_g = (sem_g0, sem_g1)

        def zrow(i, carry):
            for g in range(HALF // 16):
                hbuf0[i, pl.ds(g * 16, 16)] = jnp.zeros((16,), jnp.float32)
            return carry

        lax.fori_loop(0, E_CH, zrow, 0)
        row0 = s * FLUSH_ROWS

        @pl.when(s < FLUSH_TILES)
        def _():
            for part in range(FLUSH_ROWS // E_CH):
                pltpu.sync_copy(hbuf0,
                                s_sh.at[pl.ds(row0 + part * E_CH, E_CH), :])
            pltpu.sync_copy(
                hbuf0.at[pl.ds(0, E_H), :],
                s_sh.at[pl.ds(row0 + (FLUSH_ROWS // E_CH) * E_CH, E_H), :])

        plsc.subcore_barrier()

        base = s * EDGES_PER_TILE

        def edge_loop(c_hbm, h1_hbm):
            def issue_idx(kch, p):
                e0 = base + kch * E_CH
                pltpu.async_copy(src_h.at[pl.ds(e0, E_CH)], sidx[p], sem_si[p])
                pltpu.async_copy(dst_h.at[pl.ds(e0, E_CH)], didx[p], sem_di[p])

            def wait_idx(p):
                pltpu.make_async_copy(src_h.at[pl.ds(0, E_CH)], sidx[p],
                                      sem_si[p]).wait()
                pltpu.make_async_copy(dst_h.at[pl.ds(0, E_CH)], didx[p],
                                      sem_di[p]).wait()

            def issue_data(kch, p):
                e0 = base + kch * E_CH
                pltpu.async_copy(c_hbm.at[pl.ds(e0, E_CH), :], cbufs[p],
                                 sem_c[p])
                pltpu.async_copy(h1_hbm.at[didx[p]], hbufs[p], sem_g[p])

            def wait_data(p):
                pltpu.make_async_copy(c_hbm.at[pl.ds(0, E_CH), :], cbufs[p],
                                      sem_c[p]).wait()
                pltpu.make_async_copy(h1_hbm.at[didx[p]], hbufs[p],
                                      sem_g[p]).wait()

            issue_idx(0, 0)
            wait_idx(0)
            issue_data(0, 0)
            issue_idx(1, 1)

            def body(i, carry):
                for p in range(2):
                    kch = 2 * i + p
                    q = 1 - p
                    cbuf = cbufs[p]
                    hbuf = hbufs[p]

                    @pl.when(kch + 1 < N_CHUNKS)
                    def _():
                        wait_idx(q)

                        @pl.when(kch >= 1)
                        def _():
                            pltpu.make_async_copy(hbufs[q],
                                                  s_sh.at[sidx[q]],
                                                  sem_s[q]).wait()

                        issue_data(kch + 1, q)

                    wait_data(p)

                    def row(r, carry2):
                        cws = [cbuf[r, pl.ds(g * 16, 16)]
                               for g in range(HALF // 32)]
                        hlos = [hbuf[r, pl.ds(g * 16, 16)]
                                for g in range(HALF // 32)]
                        hhis = [hbuf[r, pl.ds(64 + g * 16, 16)]
                                for g in range(HALF // 32)]
                        for g in range(HALF // 32):
                            lo = ((cws[g] << 16) >> 16).astype(jnp.float32)
                            hi = (cws[g] >> 16).astype(jnp.float32)
                            hbuf[r, pl.ds(g * 32, 16)] = jnp.maximum(
                                lo + hlos[g], 0.0)
                            hbuf[r, pl.ds(g * 32 + 16, 16)] = jnp.maximum(
                                hi + hhis[g], 0.0)
                        return carry2

                    lax.fori_loop(0, E_CH, row, 0)
                    pltpu.async_copy(hbuf, s_sh.at[sidx[p]], sem_s[p],
                                     add=True)

                    @pl.when(kch + 2 < N_CHUNKS)
                    def _():
                        issue_idx(kch + 2, p)
                return carry

            lax.fori_loop(0, N_CHUNKS // 2, body, 0)
            for p in range(2):
                pltpu.make_async_copy(hbufs[p], s_sh.at[sidx[p]],
                                      sem_s[p]).wait()

        @pl.when(c == 0)
        def _():
            edge_loop(ca_h, h1a_h)

        @pl.when(c == 1)
        def _():
            edge_loop(cb_h, h1b_h)

        plsc.subcore_barrier()

        @pl.when(jnp.logical_and(c == 0, s < FLUSH_TILES))
        def _():
            pltpu.sync_copy(s_sh.at[pl.ds(row0, FLUSH_ROWS), :],
                            sa_h.at[pl.ds(row0, FLUSH_ROWS), :])

        @pl.when(jnp.logical_and(c == 1, s < FLUSH_TILES))
        def _():
            pltpu.sync_copy(s_sh.at[pl.ds(row0, FLUSH_ROWS), :],
                            sb_h.at[pl.ds(row0, FLUSH_ROWS), :])

    return k(src, dst, ca, cb, h1a, h1b)


def _update_body(sa_ref, sb_ref, h_ref, deg_ref, w2a_ref, w2b_ref, b2_ref,
                 wih_ref, bih_ref, whh_ref, bhh_ref, out_ref):
    deg = jnp.sum(deg_ref[...], axis=1, keepdims=True)
    m_v = (jnp.dot(sa_ref[...], w2a_ref[...], preferred_element_type=jnp.float32)
           + jnp.dot(sb_ref[...], w2b_ref[...], preferred_element_type=jnp.float32)
           + deg * b2_ref[...])
    h = h_ref[...]
    gi = lax.dot_general(m_v, wih_ref[...], (((1,), (1,)), ((), ())),
                         preferred_element_type=jnp.float32) + bih_ref[...]
    gh = lax.dot_general(h, whh_ref[...], (((1,), (1,)), ((), ())),
                         preferred_element_type=jnp.float32) + bhh_ref[...]
    ir, iz, inn = gi[:, :N_DIM], gi[:, N_DIM:2 * N_DIM], gi[:, 2 * N_DIM:]
    hr, hz, hn = gh[:, :N_DIM], gh[:, N_DIM:2 * N_DIM], gh[:, 2 * N_DIM:]
    r = jax.nn.sigmoid(ir + hr)
    z = jax.nn.sigmoid(iz + hz)
    n = jnp.tanh(inn + r * hn)
    out_ref[...] = (1.0 - z) * n + z * h


def _update(sa, sb, h_v, deg2, w2a, w2b, b2, wih, bih, whh, bhh):
    full = lambda r, c: pl.BlockSpec((r, c), lambda i: (0, 0))
    return pl.pallas_call(
        _update_body,
        grid=(N_NODE // NB,),
        in_specs=[
            pl.BlockSpec((NB, HALF), lambda i: (i, 0)),
            pl.BlockSpec((NB, HALF), lambda i: (i, 0)),
            pl.BlockSpec((NB, N_DIM), lambda i: (i, 0)),
            pl.BlockSpec((NB, 16), lambda i: (i, 0)),
            full(HALF, M_DIM), full(HALF, M_DIM), full(1, M_DIM),
            full(3 * N_DIM, M_DIM), full(1, 3 * N_DIM),
            full(3 * N_DIM, N_DIM), full(1, 3 * N_DIM),
        ],
        out_specs=pl.BlockSpec((NB, N_DIM), lambda i: (i, 0)),
        out_shape=jax.ShapeDtypeStruct((N_NODE, N_DIM), jnp.float32),
    )(sa, sb, h_v, deg2, w2a, w2b, b2, wih, bih, whh, bhh)


def _readout_body(h_ref, h0_ref, g_ref, riw1_ref, rib1_ref, riw2_ref, rib2_ref,
                  rjw1_ref, rjb1_ref, rjw2_ref, rjb2_ref,
                  cw1_ref, cb1_ref, cw2_ref, cb2_ref, out_ref, racc):
    i = pl.program_id(0)

    @pl.when(i == 0)
    def _():
        racc[...] = jnp.zeros_like(racc)

    h = h_ref[...]
    cat = jnp.concatenate([h, h0_ref[...]], axis=1)
    hi = jnp.maximum(jnp.dot(cat, riw1_ref[...],
                             preferred_element_type=jnp.float32) + rib1_ref[...], 0.0)
    i_out = jax.nn.sigmoid(jnp.dot(hi, riw2_ref[...],
                                   preferred_element_type=jnp.float32) + rib2_ref[...])
    hj = jnp.maximum(jnp.dot(h, rjw1_ref[...],
                             preferred_element_type=jnp.float32) + rjb1_ref[...], 0.0)
    j_out = jnp.dot(hj, rjw2_ref[...],
                    preferred_element_type=jnp.float32) + rjb2_ref[...]
    p = i_out * j_out
    cols = lax.broadcasted_iota(jnp.int32, (NB, N_GRAPH), 1)
    onehot = (cols == g_ref[...]).astype(jnp.float32)
    racc[...] += lax.dot_general(onehot, p, (((0,), (0,)), ((), ())),
                                 preferred_element_type=jnp.float32)

    @pl.when(i == N_NODE // NB - 1)
    def _():
        rg = racc[...]
        hc = jnp.maximum(jnp.dot(rg, cw1_ref[...],
                                 preferred_element_type=jnp.float32) + cb1_ref[...], 0.0)
        logit = jnp.dot(hc, cw2_ref[...],
                        preferred_element_type=jnp.float32) + cb2_ref[...]
        out_ref[...] = jax.nn.sigmoid(logit)


def _readout(h_v, h_0, gidx2, riw1, rib1, riw2, rib2, rjw1, rjb1, rjw2, rjb2,
             cw1, cb1, cw2, cb2):
    full = lambda r, c: pl.BlockSpec((r, c), lambda i: (0, 0))
    return pl.pallas_call(
        _readout_body,
        grid=(N_NODE // NB,),
        in_specs=[
            pl.BlockSpec((NB, N_DIM), lambda i: (i, 0)),
            pl.BlockSpec((NB, N_DIM), lambda i: (i, 0)),
            pl.BlockSpec((NB, 1), lambda i: (i, 0)),
            full(2 * N_DIM, 256), full(1, 256), full(256, G_DIM), full(1, G_DIM),
            full(N_DIM, 256), full(1, 256), full(256, G_DIM), full(1, G_DIM),
            full(G_DIM, 128), full(1, 128), full(128, 1), full(1, 1),
        ],
        out_specs=pl.BlockSpec((N_GRAPH, 1), lambda i: (0, 0)),
        out_shape=jax.ShapeDtypeStruct((N_GRAPH, 1), jnp.float32),
        scratch_shapes=[pltpu.VMEM((N_GRAPH, G_DIM), jnp.float32)],
    )(h_v, h_0, gidx2, riw1, rib1, riw2, rib2, rjw1, rjb1, rjw2, rjb2,
      cw1, cb1, cw2, cb2)


def kernel(h_0, E_attr, m_W1, m_b1, m_W2, m_b2, gru_Wih, gru_Whh, gru_bih,
           gru_bhh, ri_W1, ri_b1, ri_W2, ri_b2, rj_W1, rj_b1, rj_W2, rj_b2,
           c_W1, c_b1, c_W2, c_b2, graph_index, E):
    src = E[0]
    dst = E[1]
    w1h = m_W1[:N_DIM]
    w1e = m_W1[N_DIM:]

    ca, cb = _build_c(E_attr, w1e, m_b1.reshape(1, -1))
    deg2 = _deg_sc(src)

    pi = []
    for g in range(HALF // 32):
        pi.extend(16 * g + i for i in range(16))
        pi.extend(64 + 16 * g + i for i in range(16))
    pi = jnp.array(pi, dtype=jnp.int32)
    w2a = m_W2[:HALF][pi] * (1.0 / QSCALE)
    w2b = m_W2[HALF:][pi] * (1.0 / QSCALE)
    b2 = m_b2.reshape(1, -1)
    bih = gru_bih.reshape(1, -1)
    bhh = gru_bhh.reshape(1, -1)

    h_v = h_0
    for _ in range(T):
        h1a, h1b = _build_h1(h_v, w1h)
        sa, sb = _message_sc(src, dst, ca, cb, h1a, h1b)
        h_v = _update(sa, sb, h_v, deg2, w2a, w2b, b2, gru_Wih, bih,
                      gru_Whh, bhh)

    out = _readout(h_v, h_0, graph_index.reshape(N_NODE, 1),
                   ri_W1, ri_b1.reshape(1, -1), ri_W2, ri_b2.reshape(1, -1),
                   rj_W1, rj_b1.reshape(1, -1), rj_W2, rj_b2.reshape(1, -1),
                   c_W1, c_b1.reshape(1, -1), c_W2, c_b2.reshape(1, -1))
    return out.reshape(N_GRAPH)
